# baseline (speedup 1.0000x reference)
import functools

import jax
import jax.numpy as jnp
from jax.experimental import pallas as pl

_dot = functools.partial(jnp.dot, preferred_element_type=jnp.float32,
                         precision=jax.lax.Precision.HIGHEST)

B = 65536
H = 64
NL = 3
NS = 2
NR = 4
NE = NS + NR
BLK = 2048


def _moe_kernel(xt_ref, in_Wt_ref, in_b_ref, Wcat1_ref, bcat1_ref,
                W2stk_ref, b2sh_ref, rt_b2_ref, rtr_Wt_ref, rtr_b_ref,
                sel_ref, out_Wt_ref, out_b_ref, u_ref):
    f32 = jnp.float32
    xt = xt_ref[...]
    h = jnp.tanh(_dot(xt, in_Wt_ref[...])
                 + in_b_ref[...])
    sel = sel_ref[...]
    for l in range(NL):
        a = jnp.tanh(_dot(h, Wcat1_ref[l])
                     + bcat1_ref[l])
        logits = (_dot(h, rtr_Wt_ref[l])
                  + rtr_b_ref[l])
        m = jnp.max(logits, axis=1, keepdims=True)
        e = jnp.exp(logits - m)
        rw = e / jnp.sum(e, axis=1, keepdims=True)
        c = [rw[:, i:i + 1] for i in range(NR)]
        masks = []
        for i in range(NR):
            rank = jnp.zeros_like(c[i])
            for j in range(NR):
                if j == i:
                    continue
                beats = (c[j] > c[i]) if j > i else (c[j] >= c[i])
                rank = rank + beats.astype(f32)
            masks.append((rank < 2.0).astype(f32))
        rwm = jnp.concatenate(masks, axis=1) * rw
        scale_rt = _dot(rwm, sel)
        a = jnp.concatenate([a[:, :NS * H], a[:, NS * H:] * scale_rt], axis=1)
        out2 = _dot(a, W2stk_ref[l])
        bias = b2sh_ref[l] + _dot(rwm, rt_b2_ref[l])
        h = jnp.tanh(h + out2 + bias)
    u_ref[...] = (_dot(h, out_Wt_ref[...])
                  + out_b_ref[...])


@jax.jit
def kernel(x, t, in_W, in_b, sh_W1, sh_b1, sh_W2, sh_b2,
           rt_W1, rt_b1, rt_W2, rt_b2, rtr_W, rtr_b, out_W, out_b):
    xt = jnp.concatenate([x, t], axis=1)
    W1all = jnp.concatenate([sh_W1, rt_W1], axis=1)
    b1all = jnp.concatenate([sh_b1, rt_b1], axis=1)
    W2all = jnp.concatenate([sh_W2, rt_W2], axis=1)
    Wcat1 = jnp.transpose(W1all, (0, 1, 3, 2)).transpose(0, 2, 1, 3)
    Wcat1 = Wcat1.reshape(NL, H, NE * H)
    bcat1 = b1all.reshape(NL, NE * H)
    W2stk = jnp.transpose(W2all, (0, 1, 3, 2)).reshape(NL, NE * H, H)
    b2sh = jnp.sum(sh_b2, axis=1)
    rtr_Wt = jnp.transpose(rtr_W, (0, 2, 1))
    sel = jnp.repeat(jnp.eye(NR, dtype=jnp.float32), H, axis=1)
    in_Wt = in_W.T
    out_Wt = out_W.T
    in_b2 = in_b.reshape(1, H)
    out_b2 = out_b.reshape(1, 1)

    grid = (B // BLK,)
    u = pl.pallas_call(
        _moe_kernel,
        grid=grid,
        in_specs=[
            pl.BlockSpec((BLK, 2), lambda i: (i, 0)),
            pl.BlockSpec((2, H), lambda i: (0, 0)),
            pl.BlockSpec((1, H), lambda i: (0, 0)),
            pl.BlockSpec((NL, H, NE * H), lambda i: (0, 0, 0)),
            pl.BlockSpec((NL, NE * H), lambda i: (0, 0)),
            pl.BlockSpec((NL, NE * H, H), lambda i: (0, 0, 0)),
            pl.BlockSpec((NL, H), lambda i: (0, 0)),
            pl.BlockSpec((NL, NR, H), lambda i: (0, 0, 0)),
            pl.BlockSpec((NL, H, NR), lambda i: (0, 0, 0)),
            pl.BlockSpec((NL, NR), lambda i: (0, 0)),
            pl.BlockSpec((NR, NR * H), lambda i: (0, 0)),
            pl.BlockSpec((H, 1), lambda i: (0, 0)),
            pl.BlockSpec((1, 1), lambda i: (0, 0)),
        ],
        out_specs=pl.BlockSpec((BLK, 1), lambda i: (i, 0)),
        out_shape=jax.ShapeDtypeStruct((B, 1), jnp.float32),
    )(xt, in_Wt, in_b2, Wcat1, bcat1, W2stk, b2sh, rt_b2, rtr_Wt, rtr_b,
      sel, out_Wt, out_b2)
    return u


# speedup vs baseline: 1.2032x; 1.2032x over previous
import functools

import jax
import jax.numpy as jnp
from jax.experimental import pallas as pl

_dot = functools.partial(jnp.dot, preferred_element_type=jnp.float32,
                         precision=jax.lax.Precision.HIGHEST)

B = 65536
H = 64
NL = 3
NS = 2
NR = 4
NE = NS + NR
BLK = 2048


def _moe_kernel(xt_ref, in_Wt_ref, in_b_ref, Wcat1_ref, bcat1_ref,
                W2stk_ref, b2sh_ref, rt_b2_ref, rtr_Wt_ref, rtr_b_ref,
                out_Wt_ref, out_b_ref, u_ref):
    f32 = jnp.float32
    xt = xt_ref[...]
    h = jnp.tanh(_dot(xt, in_Wt_ref[...])
                 + in_b_ref[...])
    for l in range(NL):
        a = jnp.tanh(_dot(h, Wcat1_ref[l])
                     + bcat1_ref[l])
        logits = (_dot(h, rtr_Wt_ref[l])
                  + rtr_b_ref[l])
        m = jnp.max(logits, axis=1, keepdims=True)
        e = jnp.exp(logits - m)
        rw = e / jnp.sum(e, axis=1, keepdims=True)
        c = [rw[:, i:i + 1] for i in range(NR)]
        masks = []
        for i in range(NR):
            rank = jnp.zeros_like(c[i])
            for j in range(NR):
                if j == i:
                    continue
                beats = (c[j] > c[i]) if j > i else (c[j] >= c[i])
                rank = rank + beats.astype(f32)
            masks.append((rank < 2.0).astype(f32))
        rwm = [masks[i] * c[i] for i in range(NR)]
        scale_rt = jnp.concatenate(
            [jnp.broadcast_to(rwm[i], (rwm[i].shape[0], H)) for i in range(NR)],
            axis=1)
        a = jnp.concatenate([a[:, :NS * H], a[:, NS * H:] * scale_rt], axis=1)
        out2 = _dot(a, W2stk_ref[l])
        bias = b2sh_ref[l]
        for i in range(NR):
            bias = bias + rwm[i] * rt_b2_ref[l, i]
        h = jnp.tanh(h + out2 + bias)
    u_ref[...] = (_dot(h, out_Wt_ref[...])
                  + out_b_ref[...])


@jax.jit
def kernel(x, t, in_W, in_b, sh_W1, sh_b1, sh_W2, sh_b2,
           rt_W1, rt_b1, rt_W2, rt_b2, rtr_W, rtr_b, out_W, out_b):
    xt = jnp.concatenate([x, t], axis=1)
    W1all = jnp.concatenate([sh_W1, rt_W1], axis=1)
    b1all = jnp.concatenate([sh_b1, rt_b1], axis=1)
    W2all = jnp.concatenate([sh_W2, rt_W2], axis=1)
    Wcat1 = jnp.transpose(W1all, (0, 1, 3, 2)).transpose(0, 2, 1, 3)
    Wcat1 = Wcat1.reshape(NL, H, NE * H)
    bcat1 = b1all.reshape(NL, NE * H)
    W2stk = jnp.transpose(W2all, (0, 1, 3, 2)).reshape(NL, NE * H, H)
    b2sh = jnp.sum(sh_b2, axis=1)
    rtr_Wt = jnp.transpose(rtr_W, (0, 2, 1))
    in_Wt = in_W.T
    out_Wt = out_W.T
    in_b2 = in_b.reshape(1, H)
    out_b2 = out_b.reshape(1, 1)

    grid = (B // BLK,)
    u = pl.pallas_call(
        _moe_kernel,
        grid=grid,
        in_specs=[
            pl.BlockSpec((BLK, 2), lambda i: (i, 0)),
            pl.BlockSpec((2, H), lambda i: (0, 0)),
            pl.BlockSpec((1, H), lambda i: (0, 0)),
            pl.BlockSpec((NL, H, NE * H), lambda i: (0, 0, 0)),
            pl.BlockSpec((NL, NE * H), lambda i: (0, 0)),
            pl.BlockSpec((NL, NE * H, H), lambda i: (0, 0, 0)),
            pl.BlockSpec((NL, H), lambda i: (0, 0)),
            pl.BlockSpec((NL, NR, H), lambda i: (0, 0, 0)),
            pl.BlockSpec((NL, H, NR), lambda i: (0, 0, 0)),
            pl.BlockSpec((NL, NR), lambda i: (0, 0)),
            pl.BlockSpec((H, 1), lambda i: (0, 0)),
            pl.BlockSpec((1, 1), lambda i: (0, 0)),
        ],
        out_specs=pl.BlockSpec((BLK, 1), lambda i: (i, 0)),
        out_shape=jax.ShapeDtypeStruct((B, 1), jnp.float32),
    )(xt, in_Wt, in_b2, Wcat1, bcat1, W2stk, b2sh, rt_b2, rtr_Wt, rtr_b,
      out_Wt, out_b2)
    return u


# speedup vs baseline: 3.1355x; 2.6060x over previous
import functools

import jax
import jax.numpy as jnp
from jax.experimental import pallas as pl

_dot = functools.partial(jnp.dot, preferred_element_type=jnp.float32)

B = 65536
H = 64
NL = 3
NS = 2
NR = 4
NE = NS + NR
BLK = 2048


def _moe_kernel(xt_ref, in_Wt_ref, in_b_ref, sh_W1t_ref, sh_b1_ref,
                sh_W2t_ref, sh_b2_ref, rt_W1t_ref, rt_b1_ref, rt_W2t_ref,
                rt_b2_ref, rtr_Wt_ref, rtr_b_ref, out_Wt_ref, out_b_ref,
                u_ref):
    f32 = jnp.float32
    xt = xt_ref[...]
    h = jnp.tanh(_dot(xt, in_Wt_ref[...]) + in_b_ref[...])
    for l in range(NL):
        shared = jnp.zeros_like(h)
        for s in range(NS):
            a = jnp.tanh(_dot(h, sh_W1t_ref[l, s]) + sh_b1_ref[l, s])
            shared = shared + (_dot(a, sh_W2t_ref[l, s]) + sh_b2_ref[l, s])
        logits = _dot(h, rtr_Wt_ref[l]) + rtr_b_ref[l]
        m = jnp.max(logits, axis=1, keepdims=True)
        e = jnp.exp(logits - m)
        rw = e / jnp.sum(e, axis=1, keepdims=True)
        c = [rw[:, i:i + 1] for i in range(NR)]
        masks = []
        for i in range(NR):
            rank = jnp.zeros_like(c[i])
            for j in range(NR):
                if j == i:
                    continue
                beats = (c[j] > c[i]) if j > i else (c[j] >= c[i])
                rank = rank + beats.astype(f32)
            masks.append((rank < 2.0).astype(f32))
        routed = jnp.zeros_like(h)
        for i in range(NR):
            w = c[i] * masks[i]
            a = jnp.tanh(_dot(h, rt_W1t_ref[l, i]) + rt_b1_ref[l, i])
            routed = routed + w * (_dot(a, rt_W2t_ref[l, i]) + rt_b2_ref[l, i])
        h = jnp.tanh(h + shared + routed)
    u_ref[...] = _dot(h, out_Wt_ref[...]) + out_b_ref[...]


@jax.jit
def kernel(x, t, in_W, in_b, sh_W1, sh_b1, sh_W2, sh_b2,
           rt_W1, rt_b1, rt_W2, rt_b2, rtr_W, rtr_b, out_W, out_b):
    xt = jnp.concatenate([x, t], axis=1)
    sh_W1t = jnp.transpose(sh_W1, (0, 1, 3, 2))
    sh_W2t = jnp.transpose(sh_W2, (0, 1, 3, 2))
    rt_W1t = jnp.transpose(rt_W1, (0, 1, 3, 2))
    rt_W2t = jnp.transpose(rt_W2, (0, 1, 3, 2))
    rtr_Wt = jnp.transpose(rtr_W, (0, 2, 1))
    in_Wt = in_W.T
    out_Wt = out_W.T
    in_b2 = in_b.reshape(1, H)
    out_b2 = out_b.reshape(1, 1)

    grid = (B // BLK,)
    full = lambda *s: pl.BlockSpec(s, lambda i: (0,) * len(s))
    u = pl.pallas_call(
        _moe_kernel,
        grid=grid,
        in_specs=[
            pl.BlockSpec((BLK, 2), lambda i: (i, 0)),
            full(2, H),
            full(1, H),
            full(NL, NS, H, H),
            full(NL, NS, H),
            full(NL, NS, H, H),
            full(NL, NS, H),
            full(NL, NR, H, H),
            full(NL, NR, H),
            full(NL, NR, H, H),
            full(NL, NR, H),
            full(NL, H, NR),
            full(NL, NR),
            full(H, 1),
            full(1, 1),
        ],
        out_specs=pl.BlockSpec((BLK, 1), lambda i: (i, 0)),
        out_shape=jax.ShapeDtypeStruct((B, 1), jnp.float32),
    )(xt, in_Wt, in_b2, sh_W1t, sh_b1, sh_W2t, sh_b2,
      rt_W1t, rt_b1, rt_W2t, rt_b2, rtr_Wt, rtr_b, out_Wt, out_b2)
    return u


# speedup vs baseline: 5.6662x; 1.8071x over previous
import functools

import jax
import jax.numpy as jnp
from jax.experimental import pallas as pl

_dot = functools.partial(jnp.dot, preferred_element_type=jnp.float32)

B = 65536
H = 64
NL = 3
NS = 2
NR = 4
NE = NS + NR
BLK = 2048


def _moe_kernel(xt_ref, in_Wt_ref, in_b_ref, sh_W1t_ref, sh_b1_ref,
                sh_W2t_ref, sh_b2_ref, rt_W1t_ref, rt_b1_ref, rt_W2t_ref,
                rt_b2_ref, rtr_Wt_ref, rtr_b_ref, out_Wt_ref, out_b_ref,
                u_ref):
    f32 = jnp.float32
    xt = xt_ref[...]
    h = jnp.tanh(_dot(xt, in_Wt_ref[...]) + in_b_ref[...])
    for l in range(NL):
        shared = jnp.zeros_like(h)
        for s in range(NS):
            a = jnp.tanh(_dot(h, sh_W1t_ref[l, s]) + sh_b1_ref[l, s])
            shared = shared + (_dot(a, sh_W2t_ref[l, s]) + sh_b2_ref[l, s])
        logits = _dot(h, rtr_Wt_ref[l]) + rtr_b_ref[l]
        blk = logits.shape[0]
        lp = jnp.concatenate([logits, jnp.zeros_like(logits)], axis=1)
        lT = jnp.transpose(lp)
        r = [lT[i:i + 1, :] for i in range(NR)]
        m = jnp.maximum(jnp.maximum(r[0], r[1]), jnp.maximum(r[2], r[3]))
        e = [jnp.exp(r[i] - m) for i in range(NR)]
        s = ((e[0] + e[1]) + e[2]) + e[3]
        rw = [e[i] / s for i in range(NR)]
        wrow = []
        for i in range(NR):
            rank = jnp.zeros_like(rw[i])
            for j in range(NR):
                if j == i:
                    continue
                beats = (rw[j] > rw[i]) if j > i else (rw[j] >= rw[i])
                rank = rank + beats.astype(f32)
            wrow.append(rw[i] * (rank < 2.0).astype(f32))
        wT = jnp.concatenate(wrow + [jnp.zeros((NR, blk), f32)], axis=0)
        w4 = jnp.transpose(wT)
        routed = jnp.zeros_like(h)
        for i in range(NR):
            w = w4[:, i:i + 1]
            a = jnp.tanh(_dot(h, rt_W1t_ref[l, i]) + rt_b1_ref[l, i])
            routed = routed + w * (_dot(a, rt_W2t_ref[l, i]) + rt_b2_ref[l, i])
        h = jnp.tanh(h + shared + routed)
    u_ref[...] = _dot(h, out_Wt_ref[...]) + out_b_ref[...]


@jax.jit
def kernel(x, t, in_W, in_b, sh_W1, sh_b1, sh_W2, sh_b2,
           rt_W1, rt_b1, rt_W2, rt_b2, rtr_W, rtr_b, out_W, out_b):
    xt = jnp.concatenate([x, t], axis=1)
    sh_W1t = jnp.transpose(sh_W1, (0, 1, 3, 2))
    sh_W2t = jnp.transpose(sh_W2, (0, 1, 3, 2))
    rt_W1t = jnp.transpose(rt_W1, (0, 1, 3, 2))
    rt_W2t = jnp.transpose(rt_W2, (0, 1, 3, 2))
    rtr_Wt = jnp.transpose(rtr_W, (0, 2, 1))
    in_Wt = in_W.T
    out_Wt = out_W.T
    in_b2 = in_b.reshape(1, H)
    out_b2 = out_b.reshape(1, 1)

    grid = (B // BLK,)
    full = lambda *s: pl.BlockSpec(s, lambda i: (0,) * len(s))
    u = pl.pallas_call(
        _moe_kernel,
        grid=grid,
        in_specs=[
            pl.BlockSpec((BLK, 2), lambda i: (i, 0)),
            full(2, H),
            full(1, H),
            full(NL, NS, H, H),
            full(NL, NS, H),
            full(NL, NS, H, H),
            full(NL, NS, H),
            full(NL, NR, H, H),
            full(NL, NR, H),
            full(NL, NR, H, H),
            full(NL, NR, H),
            full(NL, H, NR),
            full(NL, NR),
            full(H, 1),
            full(1, 1),
        ],
        out_specs=pl.BlockSpec((BLK, 1), lambda i: (i, 0)),
        out_shape=jax.ShapeDtypeStruct((B, 1), jnp.float32),
    )(xt, in_Wt, in_b2, sh_W1t, sh_b1, sh_W2t, sh_b2,
      rt_W1t, rt_b1, rt_W2t, rt_b2, rtr_Wt, rtr_b, out_Wt, out_b2)
    return u


# speedup vs baseline: 5.6984x; 1.0057x over previous
import functools

import jax
import jax.numpy as jnp
from jax.experimental import pallas as pl

_dot = functools.partial(jnp.dot, preferred_element_type=jnp.float32)

B = 65536
H = 64
NL = 3
NS = 2
NR = 4
NE = NS + NR
BLK = 4096


def _moe_kernel(xt_ref, in_Wt_ref, in_b_ref, sh_W1t_ref, sh_b1_ref,
                sh_W2t_ref, sh_b2_ref, rt_W1t_ref, rt_b1_ref, rt_W2t_ref,
                rt_b2_ref, rtr_Wt_ref, rtr_b_ref, out_Wt_ref, out_b_ref,
                u_ref):
    f32 = jnp.float32
    xt = xt_ref[...]
    h = jnp.tanh(_dot(xt, in_Wt_ref[...]) + in_b_ref[...])
    for l in range(NL):
        shared = jnp.zeros_like(h)
        for s in range(NS):
            a = jnp.tanh(_dot(h, sh_W1t_ref[l, s]) + sh_b1_ref[l, s])
            shared = shared + (_dot(a, sh_W2t_ref[l, s]) + sh_b2_ref[l, s])
        logits = _dot(h, rtr_Wt_ref[l]) + rtr_b_ref[l]
        blk = logits.shape[0]
        lp = jnp.concatenate([logits, jnp.zeros_like(logits)], axis=1)
        lT = jnp.transpose(lp)
        r = [lT[i:i + 1, :] for i in range(NR)]
        m = jnp.maximum(jnp.maximum(r[0], r[1]), jnp.maximum(r[2], r[3]))
        e = [jnp.exp(r[i] - m) for i in range(NR)]
        s = ((e[0] + e[1]) + e[2]) + e[3]
        rw = [e[i] / s for i in range(NR)]
        wrow = []
        for i in range(NR):
            rank = jnp.zeros_like(rw[i])
            for j in range(NR):
                if j == i:
                    continue
                beats = (rw[j] > rw[i]) if j > i else (rw[j] >= rw[i])
                rank = rank + beats.astype(f32)
            wrow.append(rw[i] * (rank < 2.0).astype(f32))
        wT = jnp.concatenate(wrow + [jnp.zeros((NR, blk), f32)], axis=0)
        w4 = jnp.transpose(wT)
        routed = jnp.zeros_like(h)
        for i in range(NR):
            w = w4[:, i:i + 1]
            a = jnp.tanh(_dot(h, rt_W1t_ref[l, i]) + rt_b1_ref[l, i])
            routed = routed + w * (_dot(a, rt_W2t_ref[l, i]) + rt_b2_ref[l, i])
        h = jnp.tanh(h + shared + routed)
    u_ref[...] = _dot(h, out_Wt_ref[...]) + out_b_ref[...]


@jax.jit
def kernel(x, t, in_W, in_b, sh_W1, sh_b1, sh_W2, sh_b2,
           rt_W1, rt_b1, rt_W2, rt_b2, rtr_W, rtr_b, out_W, out_b):
    xt = jnp.concatenate([x, t], axis=1)
    sh_W1t = jnp.transpose(sh_W1, (0, 1, 3, 2))
    sh_W2t = jnp.transpose(sh_W2, (0, 1, 3, 2))
    rt_W1t = jnp.transpose(rt_W1, (0, 1, 3, 2))
    rt_W2t = jnp.transpose(rt_W2, (0, 1, 3, 2))
    rtr_Wt = jnp.transpose(rtr_W, (0, 2, 1))
    in_Wt = in_W.T
    out_Wt = out_W.T
    in_b2 = in_b.reshape(1, H)
    out_b2 = out_b.reshape(1, 1)

    grid = (B // BLK,)
    full = lambda *s: pl.BlockSpec(s, lambda i: (0,) * len(s))
    u = pl.pallas_call(
        _moe_kernel,
        grid=grid,
        in_specs=[
            pl.BlockSpec((BLK, 2), lambda i: (i, 0)),
            full(2, H),
            full(1, H),
            full(NL, NS, H, H),
            full(NL, NS, H),
            full(NL, NS, H, H),
            full(NL, NS, H),
            full(NL, NR, H, H),
            full(NL, NR, H),
            full(NL, NR, H, H),
            full(NL, NR, H),
            full(NL, H, NR),
            full(NL, NR),
            full(H, 1),
            full(1, 1),
        ],
        out_specs=pl.BlockSpec((BLK, 1), lambda i: (i, 0)),
        out_shape=jax.ShapeDtypeStruct((B, 1), jnp.float32),
    )(xt, in_Wt, in_b2, sh_W1t, sh_b1, sh_W2t, sh_b2,
      rt_W1t, rt_b1, rt_W2t, rt_b2, rtr_Wt, rtr_b, out_Wt, out_b2)
    return u


# speedup vs baseline: 11.0175x; 1.9334x over previous
import functools

import jax
import jax.numpy as jnp
from jax.experimental import pallas as pl

_dot = functools.partial(jnp.dot, preferred_element_type=jnp.float32)

B = 65536
H = 64
NL = 3
NS = 2
NR = 4
NE = NS + NR
BLK = 4096
W1C = NE * H + 2 * NR


def _moe_kernel(xt_ref, in_Wt_ref, in_b_ref, Wcat1_ref, bcat1_ref,
                W2stk_ref, b2sh_ref, rt_b2p_ref, sel8_ref,
                out_Wt_ref, out_b_ref, u_ref):
    f32 = jnp.float32
    xt = xt_ref[...]
    h = jnp.tanh(_dot(xt, in_Wt_ref[...]) + in_b_ref[...])
    for l in range(NL):
        pre = _dot(h, Wcat1_ref[l]) + bcat1_ref[l]
        a = jnp.tanh(pre[:, :NE * H])
        lp = pre[:, NE * H:]
        lT = jnp.transpose(lp)
        r = [lT[i:i + 1, :] for i in range(NR)]
        m = jnp.maximum(jnp.maximum(r[0], r[1]), jnp.maximum(r[2], r[3]))
        e = [jnp.exp(r[i] - m) for i in range(NR)]
        s = ((e[0] + e[1]) + e[2]) + e[3]
        rw = [e[i] / s for i in range(NR)]
        wrow = []
        for i in range(NR):
            rank = jnp.zeros_like(rw[i])
            for j in range(NR):
                if j == i:
                    continue
                beats = (rw[j] > rw[i]) if j > i else (rw[j] >= rw[i])
                rank = rank + beats.astype(f32)
            wrow.append(rw[i] * (rank < 2.0).astype(f32))
        wT = jnp.concatenate(wrow + [jnp.zeros_like(lT[:NR])], axis=0)
        w4 = jnp.transpose(wT)
        scale = _dot(w4, sel8_ref[...])
        af = jnp.concatenate(
            [a[:, :NS * H], a[:, NS * H:] * scale], axis=1)
        out2 = _dot(af, W2stk_ref[l])
        bias = b2sh_ref[l] + _dot(w4, rt_b2p_ref[l])
        h = jnp.tanh(h + out2 + bias)
    u_ref[...] = _dot(h, out_Wt_ref[...]) + out_b_ref[...]


@jax.jit
def kernel(x, t, in_W, in_b, sh_W1, sh_b1, sh_W2, sh_b2,
           rt_W1, rt_b1, rt_W2, rt_b2, rtr_W, rtr_b, out_W, out_b):
    f32 = jnp.float32
    xt = jnp.concatenate([x, t], axis=1)
    W1all = jnp.concatenate([sh_W1, rt_W1], axis=1)
    b1all = jnp.concatenate([sh_b1, rt_b1], axis=1)
    W2all = jnp.concatenate([sh_W2, rt_W2], axis=1)
    Wc = jnp.transpose(W1all, (0, 3, 1, 2)).reshape(NL, H, NE * H)
    rtr_Wt = jnp.transpose(rtr_W, (0, 2, 1))
    Wcat1 = jnp.concatenate(
        [Wc, rtr_Wt, jnp.zeros((NL, H, NR), f32)], axis=2)
    bcat1 = jnp.concatenate(
        [b1all.reshape(NL, NE * H), rtr_b, jnp.zeros((NL, NR), f32)], axis=1)
    W2stk = jnp.transpose(W2all, (0, 1, 3, 2)).reshape(NL, NE * H, H)
    b2sh = jnp.sum(sh_b2, axis=1)
    rt_b2p = jnp.concatenate(
        [rt_b2, jnp.zeros((NL, NR, H), f32)], axis=1)
    sel8 = jnp.concatenate(
        [jnp.repeat(jnp.eye(NR, dtype=f32), H, axis=1),
         jnp.zeros((NR, NR * H), f32)], axis=0)
    in_Wt = in_W.T
    out_Wt = out_W.T
    in_b2 = in_b.reshape(1, H)
    out_b2 = out_b.reshape(1, 1)

    grid = (B // BLK,)
    full = lambda *s: pl.BlockSpec(s, lambda i: (0,) * len(s))
    u = pl.pallas_call(
        _moe_kernel,
        grid=grid,
        in_specs=[
            pl.BlockSpec((BLK, 2), lambda i: (i, 0)),
            full(2, H),
            full(1, H),
            full(NL, H, W1C),
            full(NL, W1C),
            full(NL, NE * H, H),
            full(NL, H),
            full(NL, 2 * NR, H),
            full(2 * NR, NR * H),
            full(H, 1),
            full(1, 1),
        ],
        out_specs=pl.BlockSpec((BLK, 1), lambda i: (i, 0)),
        out_shape=jax.ShapeDtypeStruct((B, 1), jnp.float32),
    )(xt, in_Wt, in_b2, Wcat1, bcat1, W2stk, b2sh, rt_b2p, sel8,
      out_Wt, out_b2)
    return u
